# baseline (speedup 1.0000x reference)
import functools

import jax
import jax.numpy as jnp
from jax import lax
from jax.experimental import pallas as pl
from jax.experimental.pallas import tpu as pltpu
from jax.experimental.pallas import tpu_sc as plsc

B = 128
S = 196
EMBED = 768
POOL = 512
TOP_K = 8
LENGTH = 8
ROWS = B * TOP_K

B_BLK = 4
N_CHUNKS = B // B_BLK
NBUF = 8


def _tc_body(x_hbm, key_ref, sim_ref, idx_ref, rsum_ref, idxf_ref,
             acc_ref, bufs_ref, sems):
    def _start(t):
        d = t % NBUF
        pltpu.make_async_copy(
            x_hbm.at[pl.ds(t * B_BLK, B_BLK)], bufs_ref.at[d],
            sems.at[d]).start()

    def _wait(t):
        d = t % NBUF
        pltpu.make_async_copy(
            x_hbm.at[pl.ds(t * B_BLK, B_BLK)], bufs_ref.at[d],
            sems.at[d]).wait()

    for t in range(NBUF):
        _start(t)
    for t in range(N_CHUNKS):
        _wait(t)
        part = jnp.sum(bufs_ref[t % NBUF], axis=1) * (1.0 / S)
        acc_ref[pl.ds(t * B_BLK, B_BLK), :] = part
        if t + NBUF < N_CHUNKS:
            _start(t + NBUF)

    def _finish():
        xm = acc_ref[...]
        xn = xm * lax.rsqrt(
            jnp.maximum(jnp.sum(xm * xm, axis=-1, keepdims=True), 1e-12))
        k = key_ref[...]
        kn = k * lax.rsqrt(
            jnp.maximum(jnp.sum(k * k, axis=-1, keepdims=True), 1e-12))
        sim = lax.dot_general(
            xn, kn, (((1,), (1,)), ((), ())),
            preferred_element_type=jnp.float32)
        sim_ref[...] = sim

        iota = lax.broadcasted_iota(jnp.int32, (B, POOL), 1)
        s = sim
        total = jnp.zeros((1, 1), jnp.float32)
        cols = []
        for _ in range(TOP_K):
            m = jnp.max(s, axis=1, keepdims=True)
            am = jnp.min(jnp.where(s == m, iota, POOL),
                         axis=1, keepdims=True)
            cols.append(am)
            total = total + jnp.sum(m, axis=(0, 1), keepdims=True)
            s = jnp.where(iota == am, -jnp.inf, s)
        idx = jnp.concatenate(cols, axis=1)
        idx_ref[...] = idx
        idxf_ref[...] = jnp.concatenate([idx] * (128 // TOP_K), axis=1)
        rsum_ref[...] = total * (1.0 / B)

    _finish()


def _tc_call(x_embed, prompt_key):
    return pl.pallas_call(
        _tc_body,
        in_specs=[pl.BlockSpec(memory_space=pltpu.HBM),
                  pl.BlockSpec(memory_space=pltpu.VMEM)],
        out_specs=[
            pl.BlockSpec(memory_space=pltpu.VMEM),
            pl.BlockSpec(memory_space=pltpu.VMEM),
            pl.BlockSpec(memory_space=pltpu.VMEM),
            pl.BlockSpec(memory_space=pltpu.VMEM),
        ],
        out_shape=[
            jax.ShapeDtypeStruct((B, POOL), jnp.float32),
            jax.ShapeDtypeStruct((B, TOP_K), jnp.int32),
            jax.ShapeDtypeStruct((1, 1), jnp.float32),
            jax.ShapeDtypeStruct((B, 128), jnp.int32),
        ],
        scratch_shapes=[
            pltpu.VMEM((B, EMBED), jnp.float32),
            pltpu.VMEM((NBUF, B_BLK, S, EMBED), jnp.float32),
            pltpu.SemaphoreType.DMA((NBUF,)),
        ],
    )(x_embed, prompt_key)


_NC, _NS = 2, 16
_NW = _NC * _NS
_B_PER_W = ROWS // _NW
_CHUNK = 8
_NCHUNK = _B_PER_W // _CHUNK
_B_OF_W = B // _NW


@functools.cache
def _sc_gather_fn():
    @functools.partial(
        pl.kernel,
        mesh=plsc.VectorSubcoreMesh(
            core_axis_name="c", subcore_axis_name="s"),
        out_type=jax.ShapeDtypeStruct((ROWS, LENGTH, EMBED), jnp.float32),
        scratch_types=[
            pltpu.VMEM((_B_OF_W, 128), jnp.int32),
            pltpu.VMEM((_B_PER_W,), jnp.int32),
            pltpu.VMEM((_CHUNK, LENGTH, EMBED), jnp.float32),
            pltpu.SemaphoreType.DMA,
        ],
        compiler_params=pltpu.CompilerParams(
            use_tc_tiling_on_sc=True, needs_layout_passes=False),
    )
    def _sc_gather(table_hbm, idx_hbm, out_hbm, idx2d_v, idx_v, rows_v, sem):
        wid = lax.axis_index("s") * _NC + lax.axis_index("c")
        base = wid * _B_PER_W
        pltpu.sync_copy(idx_hbm.at[pl.ds(wid * _B_OF_W, _B_OF_W)], idx2d_v)
        for h in range(_B_PER_W // 16):
            gi = lax.iota(jnp.int32, 16) + 16 * h
            g = plsc.load_gather(idx2d_v, [gi >> 3, gi & 7])
            idx_v[pl.ds(16 * h, 16)] = g
        for c in range(_NCHUNK):
            pltpu.async_copy(
                table_hbm.at[idx_v.at[pl.ds(c * _CHUNK, _CHUNK)]],
                rows_v, sem).wait()
            pltpu.sync_copy(
                rows_v, out_hbm.at[pl.ds(base + c * _CHUNK, _CHUNK)])

    return _sc_gather


def kernel(x_embed, prompt, prompt_key):
    sim, idx, rsum, idx_pad = _tc_call(x_embed, prompt_key)
    table = jnp.reshape(prompt, (POOL, LENGTH, EMBED))
    rows = _sc_gather_fn()(table, idx_pad)
    batched_prompt = jnp.reshape(rows, (1, B, TOP_K * LENGTH, EMBED))
    return batched_prompt, rsum[0, 0], sim, idx


# speedup vs baseline: 2.0112x; 2.0112x over previous
import functools

import jax
import jax.numpy as jnp
from jax import lax
from jax.experimental import pallas as pl
from jax.experimental.pallas import tpu as pltpu
from jax.experimental.pallas import tpu_sc as plsc

B = 128
S = 196
EMBED = 768
POOL = 512
TOP_K = 8
LENGTH = 8
ROWS = B * TOP_K

S_BLK = 28
N_STEPS = S // S_BLK


def _tc_body(x_ref, key_ref, sim_ref, idx_ref, rsum_ref, idxf_ref, acc_ref):
    i = pl.program_id(0)
    part = jnp.sum(x_ref[...], axis=0)

    @pl.when(i == 0)
    def _():
        acc_ref[...] = part

    @pl.when(i > 0)
    def _():
        acc_ref[...] = acc_ref[...] + part

    @pl.when(i == N_STEPS - 1)
    def _finish():
        xm = acc_ref[...] * (1.0 / S)
        xn = xm * lax.rsqrt(
            jnp.maximum(jnp.sum(xm * xm, axis=-1, keepdims=True), 1e-12))
        k = key_ref[...]
        kn = k * lax.rsqrt(
            jnp.maximum(jnp.sum(k * k, axis=-1, keepdims=True), 1e-12))
        sim = lax.dot_general(
            xn, kn, (((1,), (1,)), ((), ())),
            preferred_element_type=jnp.float32)
        sim_ref[...] = sim

        iota = lax.broadcasted_iota(jnp.int32, (B, POOL), 1)
        s = sim
        total = jnp.zeros((1, 1), jnp.float32)
        cols = []
        for _ in range(TOP_K):
            m = jnp.max(s, axis=1, keepdims=True)
            am = jnp.min(jnp.where(s == m, iota, POOL),
                         axis=1, keepdims=True)
            cols.append(am)
            total = total + jnp.sum(m, axis=(0, 1), keepdims=True)
            s = jnp.where(iota == am, -jnp.inf, s)
        idx = jnp.concatenate(cols, axis=1)
        idx_ref[...] = idx
        idxf_ref[...] = jnp.concatenate([idx] * (128 // TOP_K), axis=1)
        rsum_ref[...] = total * (1.0 / B)


def _tc_call(x_embed, prompt_key):
    xt = jnp.transpose(x_embed, (1, 0, 2))
    return pl.pallas_call(
        _tc_body,
        grid=(N_STEPS,),
        in_specs=[
            pl.BlockSpec((S_BLK, B, EMBED), lambda i: (i, 0, 0)),
            pl.BlockSpec((POOL, EMBED), lambda i: (0, 0)),
        ],
        out_specs=[
            pl.BlockSpec((B, POOL), lambda i: (0, 0)),
            pl.BlockSpec((B, TOP_K), lambda i: (0, 0)),
            pl.BlockSpec((1, 1), lambda i: (0, 0)),
            pl.BlockSpec((B, 128), lambda i: (0, 0)),
        ],
        out_shape=[
            jax.ShapeDtypeStruct((B, POOL), jnp.float32),
            jax.ShapeDtypeStruct((B, TOP_K), jnp.int32),
            jax.ShapeDtypeStruct((1, 1), jnp.float32),
            jax.ShapeDtypeStruct((B, 128), jnp.int32),
        ],
        scratch_shapes=[pltpu.VMEM((B, EMBED), jnp.float32)],
    )(xt, prompt_key)


_NC, _NS = 2, 16
_NW = _NC * _NS
_B_PER_W = ROWS // _NW
_CHUNK = 8
_NCHUNK = _B_PER_W // _CHUNK
_B_OF_W = B // _NW


@functools.cache
def _sc_gather_fn():
    @functools.partial(
        pl.kernel,
        mesh=plsc.VectorSubcoreMesh(
            core_axis_name="c", subcore_axis_name="s"),
        out_type=jax.ShapeDtypeStruct((ROWS, LENGTH, EMBED), jnp.float32),
        scratch_types=[
            pltpu.VMEM((_B_OF_W, 128), jnp.int32),
            pltpu.VMEM((_B_PER_W,), jnp.int32),
            pltpu.VMEM((_CHUNK, LENGTH, EMBED), jnp.float32),
            pltpu.SemaphoreType.DMA,
        ],
        compiler_params=pltpu.CompilerParams(
            use_tc_tiling_on_sc=True, needs_layout_passes=False),
    )
    def _sc_gather(table_hbm, idx_hbm, out_hbm, idx2d_v, idx_v, rows_v, sem):
        wid = lax.axis_index("s") * _NC + lax.axis_index("c")
        base = wid * _B_PER_W
        pltpu.sync_copy(idx_hbm.at[pl.ds(wid * _B_OF_W, _B_OF_W)], idx2d_v)
        for h in range(_B_PER_W // 16):
            gi = lax.iota(jnp.int32, 16) + 16 * h
            g = plsc.load_gather(idx2d_v, [gi >> 3, gi & 7])
            idx_v[pl.ds(16 * h, 16)] = g
        for c in range(_NCHUNK):
            pltpu.async_copy(
                table_hbm.at[idx_v.at[pl.ds(c * _CHUNK, _CHUNK)]],
                rows_v, sem).wait()
            pltpu.sync_copy(
                rows_v, out_hbm.at[pl.ds(base + c * _CHUNK, _CHUNK)])

    return _sc_gather


def kernel(x_embed, prompt, prompt_key):
    sim, idx, rsum, idx_pad = _tc_call(x_embed, prompt_key)
    table = jnp.reshape(prompt, (POOL, LENGTH, EMBED))
    rows = _sc_gather_fn()(table, idx_pad)
    batched_prompt = jnp.reshape(rows, (1, B, TOP_K * LENGTH, EMBED))
    return batched_prompt, rsum[0, 0], sim, idx
